# baseline (speedup 1.0000x reference)
import functools

import jax
import jax.numpy as jnp
from jax import lax
from jax.experimental import pallas as pl
from jax.experimental.pallas import tpu as pltpu
from jax.experimental.pallas import tpu_sc as plsc

B, S, D, H, E, K, V = 1, 2048, 768, 8, 8, 2, 100000
DH = D // H
F = 4 * D
BS = 128
PAD_LEN = S * K + E * BS
NB = PAD_LEN // BS
NW = 32


def _sc_gather(table, idx):
    n = idx.shape[0]
    d = table.shape[1]
    bpw = n // NW
    chunk = bpw if bpw <= 128 else bpw // 2
    nchunk = bpw // chunk
    mesh = plsc.VectorSubcoreMesh(core_axis_name="c", subcore_axis_name="s")

    @functools.partial(
        pl.kernel, mesh=mesh,
        out_type=jax.ShapeDtypeStruct((n, d), jnp.float32),
        scratch_types=[
            pltpu.VMEM((chunk,), jnp.int32),
            pltpu.VMEM((chunk, d), jnp.float32),
            pltpu.SemaphoreType.DMA,
        ],
    )
    def k(table_hbm, idx_hbm, out_hbm, idx_v, rows_v, sem):
        wid = lax.axis_index("s") * 2 + lax.axis_index("c")
        base = wid * bpw
        for c in range(nchunk):
            off = base + c * chunk
            pltpu.sync_copy(idx_hbm.at[pl.ds(off, chunk)], idx_v)
            pltpu.async_copy(table_hbm.at[idx_v], rows_v, sem).wait()
            pltpu.sync_copy(rows_v, out_hbm.at[pl.ds(off, chunk)])

    return k(table, idx)


def _sc_combine(gy, pos0, pos1):
    bpw = S // NW
    nc = D // 16
    mesh = plsc.VectorSubcoreMesh(core_axis_name="c", subcore_axis_name="s")

    @functools.partial(
        pl.kernel, mesh=mesh,
        out_type=jax.ShapeDtypeStruct((S, D), jnp.float32),
        scratch_types=[
            pltpu.VMEM((bpw,), jnp.int32),
            pltpu.VMEM((bpw,), jnp.int32),
            pltpu.VMEM((bpw, D), jnp.float32),
            pltpu.VMEM((bpw, D), jnp.float32),
            pltpu.SemaphoreType.DMA,
        ],
    )
    def k(gy_hbm, p0_hbm, p1_hbm, out_hbm, i0, i1, b0, b1, sem):
        wid = lax.axis_index("s") * 2 + lax.axis_index("c")
        base = wid * bpw
        pltpu.sync_copy(p0_hbm.at[pl.ds(base, bpw)], i0)
        pltpu.sync_copy(p1_hbm.at[pl.ds(base, bpw)], i1)
        pltpu.async_copy(gy_hbm.at[i0], b0, sem).wait()
        pltpu.async_copy(gy_hbm.at[i1], b1, sem).wait()

        def row(r, _):
            for c in range(nc):
                sl = pl.ds(c * 16, 16)
                b0[r, sl] = b0[r, sl] + b1[r, sl]
            return 0

        lax.fori_loop(0, bpw, row, 0)
        pltpu.sync_copy(b0, out_hbm.at[pl.ds(base, bpw)])

    return k(gy, pos0, pos1)


def _bf(t):
    return t.astype(jnp.bfloat16)


def _moe_body(be_ref, gx_ref, w1_ref, b1_ref, w2_ref, b2_ref, gg_ref, o_ref):
    h = jnp.maximum(
        jnp.dot(_bf(gx_ref[...]), _bf(w1_ref[0]),
                preferred_element_type=jnp.float32)
        + b1_ref[0], 0.0)
    y = (jnp.dot(_bf(h), _bf(w2_ref[0]), preferred_element_type=jnp.float32)
         + b2_ref[0])
    o_ref[...] = y * gg_ref[0, 0][:, None]


def _moe_mlp(gx, w1, b1, w2, b2, ggate, bexp):
    grid_spec = pltpu.PrefetchScalarGridSpec(
        num_scalar_prefetch=1,
        grid=(NB,),
        in_specs=[
            pl.BlockSpec((BS, D), lambda i, be: (i, 0)),
            pl.BlockSpec((1, D, F), lambda i, be: (be[i], 0, 0)),
            pl.BlockSpec((1, 1, F), lambda i, be: (be[i], 0, 0)),
            pl.BlockSpec((1, F, D), lambda i, be: (be[i], 0, 0)),
            pl.BlockSpec((1, 1, D), lambda i, be: (be[i], 0, 0)),
            pl.BlockSpec((1, 1, BS), lambda i, be: (i, 0, 0)),
        ],
        out_specs=pl.BlockSpec((BS, D), lambda i, be: (i, 0)),
    )
    return pl.pallas_call(
        _moe_body,
        grid_spec=grid_spec,
        out_shape=jax.ShapeDtypeStruct((PAD_LEN, D), jnp.float32),
    )(bexp, gx, w1, b1.reshape(E, 1, F), w2, b2.reshape(E, 1, D),
      ggate.reshape(NB, 1, BS))


def _meancls_body(y_ref, wc_ref, bc_ref, f_ref, c_ref):
    f = jnp.sum(y_ref[...], axis=0, keepdims=True) * (1.0 / S)
    f_ref[...] = f
    c_ref[...] = jnp.dot(_bf(f), _bf(wc_ref[...]),
                         preferred_element_type=jnp.float32) + bc_ref[...]


def _meancls(y, wc, bc):
    return pl.pallas_call(
        _meancls_body,
        grid=(1,),
        in_specs=[pl.BlockSpec((S, D), lambda i: (0, 0)),
                  pl.BlockSpec((D, 10), lambda i: (0, 0)),
                  pl.BlockSpec((1, 10), lambda i: (0, 0))],
        out_specs=[pl.BlockSpec((1, D), lambda i: (0, 0)),
                   pl.BlockSpec((1, 10), lambda i: (0, 0))],
        out_shape=[jax.ShapeDtypeStruct((1, D), jnp.float32),
                   jax.ShapeDtypeStruct((1, 10), jnp.float32)],
    )(y, wc, bc.reshape(1, 10))


def _layernorm(x, g, b):
    m = x.mean(-1, keepdims=True)
    v = ((x - m) ** 2).mean(-1, keepdims=True)
    return (x - m) / jnp.sqrt(v + 1e-5) * g + b


def _mha(x, p):
    q = (x @ p['Wq']).reshape(B, S, H, DH).transpose(0, 2, 1, 3)
    k = (x @ p['Wk']).reshape(B, S, H, DH).transpose(0, 2, 1, 3)
    v = (x @ p['Wv']).reshape(B, S, H, DH).transpose(0, 2, 1, 3)
    scores = (q @ k.transpose(0, 1, 3, 2)) / jnp.sqrt(jnp.float32(DH))
    mask = jnp.tril(jnp.ones((S, S), dtype=bool))
    scores = jnp.where(mask[None, None, :, :], scores, jnp.float32(-1e9))
    a = jax.nn.softmax(scores, axis=-1)
    o = (a @ v).transpose(0, 2, 1, 3).reshape(B, S, D)
    return o @ p['Wo']


def _route_plan(ti, g):
    flat_e = ti.reshape(-1)
    oh = (flat_e[:, None] == jnp.arange(E, dtype=ti.dtype)).astype(jnp.int32)
    ranks = jnp.cumsum(oh, axis=0) - oh
    rank = jnp.sum(ranks * oh, axis=-1)
    cnt = jnp.sum(oh, axis=0)
    padded = ((cnt + BS - 1) // BS) * BS
    ends = jnp.cumsum(padded)
    off = ends - padded
    dst = (off[flat_e] + rank).astype(jnp.int32)
    tok = jnp.repeat(jnp.arange(S, dtype=jnp.int32), K)
    gtok = jnp.zeros((PAD_LEN,), jnp.int32).at[dst].set(tok)
    ggate = jnp.zeros((PAD_LEN,), jnp.float32).at[dst].set(g.reshape(-1))
    blk_start = jnp.arange(NB, dtype=jnp.int32) * BS
    bexp = jnp.minimum(
        jnp.searchsorted(ends, blk_start, side='right'), E - 1
    ).astype(jnp.int32)
    pos = dst.reshape(S, K)
    return gtok, ggate, bexp, pos[:, 0], pos[:, 1]


def _moe_layer(x, params, ln, pref):
    xn = _layernorm(x, params[ln + '_g'], params[ln + '_b']).reshape(-1, D)
    logits = xn @ params[pref + '_Wg']
    topv, topi = jax.lax.top_k(logits, K)
    sparse = jnp.full_like(logits, -1e9)
    sparse = sparse.at[jnp.arange(S)[:, None], topi].set(topv)
    gates = jax.nn.softmax(sparse, axis=-1)
    g = jnp.take_along_axis(gates, topi, axis=1)

    gtok, ggate, bexp, pos0, pos1 = _route_plan(topi, g)
    gx = _sc_gather(xn, gtok)
    gy = _moe_mlp(gx, params[pref + '_W1'], params[pref + '_b1'],
                  params[pref + '_W2'], params[pref + '_b2'], ggate, bexp)
    return _sc_combine(gy, pos0, pos1)


def kernel(input_ids, attention_mask, params):
    ids = input_ids.reshape(-1).astype(jnp.int32)
    x0 = _sc_gather(params['emb'], ids)
    x = x0.reshape(B, S, D)
    x = x + _mha(_layernorm(x, params['ln1_g'], params['ln1_b']), params)
    x = x + params['pos']
    first = _moe_layer(x, params, 'ln2', 'moe1')
    second = _moe_layer(x, params, 'ln3', 'moe2')
    feat, cls = _meancls(second, params['Wc'], params['bc'])
    return (first.reshape(B, S, D), second.reshape(B, S, D),
            feat.reshape(B, D), cls.reshape(B, 10))
